# baseline (speedup 1.0000x reference)
import functools

import jax
import jax.numpy as jnp
from jax import lax
from jax.experimental import pallas as pl
from jax.experimental.pallas import tpu as pltpu
from jax.experimental.pallas import tpu_sc as plsc

N = 10000
NP = 10240
NPT = NP // 16
E = 320000
ET = E + N
EPAD = 331776
H = 8
C = 128
CH = 64
EPT1 = EPAD // 16
NCH1 = EPT1 // CH
EPT2 = EPAD // 32
NCH2 = EPT2 // CH
NB = NP // 256
NEG = 0.2
A_LIN = (1.0 + NEG) / 2.0
A_ABS = (1.0 - NEG) / 2.0

_mesh = plsc.VectorSubcoreMesh(core_axis_name="c", subcore_axis_name="s")


def _pack_bf16_pairs(x):
    xb = lax.bitcast_convert_type(x, jnp.uint32)
    rne = lambda b: (b + 0x7FFF + ((b >> 16) & 1)) >> 16
    lo = rne(xb[:, :C // 2])
    hi = rne(xb[:, C // 2:])
    return lax.bitcast_convert_type(lo | (hi << 16), jnp.int32)


def _proj1_body(x_ref, wl_ref, wr_ref, bl_ref, br_ref, xl_ref, xr_ref):
    x = x_ref[...]
    xl = jnp.dot(x, wl_ref[0], preferred_element_type=jnp.float32) + bl_ref[0]
    xr = jnp.dot(x, wr_ref[0], preferred_element_type=jnp.float32) + br_ref[0]
    xl_ref[...] = _pack_bf16_pairs(xl)
    xr_ref[...] = _pack_bf16_pairs(xr)


def _proj1(xp, wl3, wr3, bl3, br3):
    return pl.pallas_call(
        _proj1_body,
        grid=(H, NB),
        in_specs=[
            pl.BlockSpec((256, C), lambda h, n: (n, 0)),
            pl.BlockSpec((1, C, C), lambda h, n: (h, 0, 0)),
            pl.BlockSpec((1, C, C), lambda h, n: (h, 0, 0)),
            pl.BlockSpec((1, 1, C), lambda h, n: (h, 0, 0)),
            pl.BlockSpec((1, 1, C), lambda h, n: (h, 0, 0)),
        ],
        out_specs=[
            pl.BlockSpec((256, C // 2), lambda h, n: (h * NB + n, 0)),
            pl.BlockSpec((256, C // 2), lambda h, n: (h * NB + n, 0)),
        ],
        out_shape=[
            jax.ShapeDtypeStruct((H * NP, C // 2), jnp.int32),
            jax.ShapeDtypeStruct((H * NP, C // 2), jnp.int32),
        ],
    )(xp, wl3, wr3, bl3, br3)


def _proj2_body(h1_ref, b1_ref, wl_ref, wr_ref, bl_ref, br_ref, xl_ref, xr_ref):
    h = pl.program_id(1)
    t = jnp.maximum(h1_ref[...] + b1_ref[0], 0.0)
    cl = jnp.dot(t, wl_ref[0], preferred_element_type=jnp.float32)
    cr = jnp.dot(t, wr_ref[0], preferred_element_type=jnp.float32)

    @pl.when(h == 0)
    def _():
        xl_ref[...] = cl + bl_ref[...]
        xr_ref[...] = cr + br_ref[...]

    @pl.when(h != 0)
    def _():
        xl_ref[...] += cl
        xr_ref[...] += cr


def _proj2_cast_body(xl_ref, xr_ref, xlb_ref, xrb_ref):
    xlb_ref[...] = _pack_bf16_pairs(xl_ref[...])
    xrb_ref[...] = _pack_bf16_pairs(xr_ref[...])


def _proj2(h1f, b13, wl23, wr23, bl2r, br2r):
    xl2, xr2 = pl.pallas_call(
        _proj2_body,
        grid=(NB, H),
        in_specs=[
            pl.BlockSpec((256, C), lambda n, h: (h * NB + n, 0)),
            pl.BlockSpec((1, 1, C), lambda n, h: (h, 0, 0)),
            pl.BlockSpec((1, C, C), lambda n, h: (h, 0, 0)),
            pl.BlockSpec((1, C, C), lambda n, h: (h, 0, 0)),
            pl.BlockSpec((1, C), lambda n, h: (0, 0)),
            pl.BlockSpec((1, C), lambda n, h: (0, 0)),
        ],
        out_specs=[
            pl.BlockSpec((256, C), lambda n, h: (n, 0)),
            pl.BlockSpec((256, C), lambda n, h: (n, 0)),
        ],
        out_shape=[
            jax.ShapeDtypeStruct((NP, C), jnp.float32),
            jax.ShapeDtypeStruct((NP, C), jnp.float32),
        ],
    )(h1f, b13, wl23, wr23, bl2r, br2r)
    return pl.pallas_call(
        _proj2_cast_body,
        out_shape=[
            jax.ShapeDtypeStruct((NP, C // 2), jnp.int32),
            jax.ShapeDtypeStruct((NP, C // 2), jnp.int32),
        ],
    )(xl2, xr2)


def _pool_body(nump_ref, denp_ref, b2_ref, batch_ref, wlin_ref, blin_ref, out_ref):
    num = nump_ref[:NP, :] + nump_ref[NP:, :]
    den = denp_ref[:, :NP] + denp_ref[:, NP:]
    b2d = batch_ref[...]
    iota64 = lax.broadcasted_iota(jnp.int32, (64, 1), 0)
    mf = (b2d == iota64).astype(jnp.float32)
    wf = mf / (den + 1e-16)
    sums = jnp.dot(wf, num, preferred_element_type=jnp.float32)
    counts = jnp.sum(mf, axis=1, keepdims=True)
    hg = sums / jnp.maximum(counts, 1.0) + b2_ref[...]
    logits = jnp.dot(hg, wlin_ref[...], preferred_element_type=jnp.float32) + blin_ref[...]
    colid = lax.broadcasted_iota(jnp.int32, (64, C), 1)
    valid = colid < 40
    m = jnp.max(jnp.where(valid, logits, -1e30), axis=1, keepdims=True)
    ex = jnp.where(valid, jnp.exp(logits - m), 0.0)
    lse = jnp.log(jnp.sum(ex, axis=1, keepdims=True))
    out_ref[...] = logits - m - lse


def _pool(numpf, denpf, b2r, batchp, wlinp, blinp):
    return pl.pallas_call(
        _pool_body,
        out_shape=jax.ShapeDtypeStruct((64, C), jnp.float32),
    )(numpf, denpf, b2r, batchp, wlinp, blinp)


def _zero_rows(ref, rows, cols):
    def body(i, _):
        for j in range(cols // 16):
            ref[i, pl.ds(j * 16, 16)] = jnp.zeros((16,), jnp.float32)
        return 0
    lax.fori_loop(0, rows, body, 0)


def _zero_flat(ref, n):
    def body(i, _):
        ref[pl.ds(i * 16, 16)] = jnp.zeros((16,), jnp.float32)
        return 0
    lax.fori_loop(0, n // 16, body, 0)


def _att_vecs(attr):
    alin, aabs = [], []
    for j in range(C // 32):
        lo = attr[0, pl.ds(j * 16, 16)]
        hi = attr[0, pl.ds(C // 2 + j * 16, 16)]
        alin.append(plsc.pack(lo * A_LIN, hi * A_LIN,
                              format=plsc.PackFormat.INTERLEAVED))
        aabs.append(plsc.pack(lo * A_ABS, hi * A_ABS,
                              format=plsc.PackFormat.INTERLEAVED))
    return alin, aabs


def _issue_idx(u, ebase, src_hbm, dst_hbm, sidx2, didx2, sem_i):
    p = lax.rem(u, 2)
    base = ebase + u * CH
    pltpu.async_copy(src_hbm.at[pl.ds(base, CH)], sidx2.at[pl.ds(p * CH, CH)],
                     sem_i.at[p])
    pltpu.async_copy(dst_hbm.at[pl.ds(base, CH)], didx2.at[pl.ds(p * CH, CH)],
                     sem_i.at[p])


def _issue_gather(u, off, src_hbm, xl_hbm, xr_hbm, sidx2, didx2, idx_sg2,
                  idx_dg2, xlb2, xrb2, sem_i, sem_g):
    p2 = lax.rem(u, 2)
    pltpu.make_async_copy(src_hbm.at[pl.ds(0, CH)],
                          sidx2.at[pl.ds(p2 * CH, CH)], sem_i.at[p2]).wait()
    pltpu.make_async_copy(src_hbm.at[pl.ds(0, CH)],
                          didx2.at[pl.ds(p2 * CH, CH)], sem_i.at[p2]).wait()
    for j in range(CH // 16):
        idx_sg2[pl.ds(p2 * CH + j * 16, 16)] = (
            sidx2[pl.ds(p2 * CH + j * 16, 16)] + off)
        idx_dg2[pl.ds(p2 * CH + j * 16, 16)] = (
            didx2[pl.ds(p2 * CH + j * 16, 16)] + off)
    pltpu.async_copy(xl_hbm.at[idx_sg2.at[pl.ds(p2 * CH, CH)]],
                     xlb2.at[pl.ds(p2 * CH, CH)], sem_g.at[p2])
    pltpu.async_copy(xr_hbm.at[idx_dg2.at[pl.ds(p2 * CH, CH)]],
                     xrb2.at[pl.ds(p2 * CH, CH)], sem_g.at[p2])


def _compute_chunk(p3, p2, alin, aabs, xlb2, xrb2, wbuf3, tbf, esc3):
    pb3 = p3 * CH
    pb2 = p2 * CH
    iota = lax.iota(jnp.int32, 16)
    zi = jnp.zeros((16,), jnp.int32)

    def grp(g, _):
        accs = [jnp.zeros((16,), jnp.float32) for _ in range(16)]
        for j in range(C // 32):
            for k in range(16):
                e = g * 16 + k
                u = plsc.bitcast(xlb2[pb2 + e, pl.ds(j * 16, 16)], jnp.bfloat16)
                v = plsc.bitcast(xrb2[pb2 + e, pl.ds(j * 16, 16)], jnp.bfloat16)
                z = u + v
                t = alin[j] * z + aabs[j] * jnp.abs(z)
                lo, hi = plsc.unpack(t, format=plsc.PackFormat.INTERLEAVED)
                accs[k] = accs[k] + lo + hi
        for k in range(16):
            plsc.store_scatter(tbf, [iota * 16 + k], accs[k])
        s16 = tbf[pl.ds(0, 16)]
        for i in range(1, 16):
            s16 = s16 + tbf[pl.ds(i * 16, 16)]
        esc3[pl.ds(pb3 + g * 16, 16)] = jnp.exp(s16)
        return 0

    lax.fori_loop(0, CH // 16, grp, 0)

    def wrow(e, _):
        we = plsc.load_gather(esc3, [zi + (pb3 + e)])
        for j in range(C // 32):
            t = plsc.bitcast(xlb2[pb2 + e, pl.ds(j * 16, 16)], jnp.bfloat16)
            lo, hi = plsc.unpack(t, format=plsc.PackFormat.INTERLEAVED)
            wbuf3[pb3 + e, pl.ds(j * 16, 16)] = lo * we
            wbuf3[pb3 + e, pl.ds(C // 2 + j * 16, 16)] = hi * we
        return 0

    lax.fori_loop(0, CH, wrow, 0)


def _wait_scatter(slot, wbuf3, idx_sc3, esc3, denS, outS, sem_sc):
    pltpu.make_async_copy(
        wbuf3.at[pl.ds(slot * CH, CH)],
        outS.at[idx_sc3.at[pl.ds(slot * CH, CH)]], sem_sc.at[slot]).wait()
    pltpu.make_async_copy(
        esc3.at[pl.ds(slot * CH, CH)],
        denS.at[idx_sc3.at[pl.ds(slot * CH, CH)]], sem_sc.at[slot]).wait()


def _edge_pass(nch, ebase, off, src_hbm, dst_hbm, xl_hbm, xr_hbm, alin, aabs,
               sidx2, didx2, idx_sg2, idx_dg2, idx_sc3, xlb2, xrb2, wbuf3,
               tbf, esc3, sem_i, sem_g, sem_sc, denS, outS):
    _issue_idx(0, ebase, src_hbm, dst_hbm, sidx2, didx2, sem_i)
    _issue_idx(1, ebase, src_hbm, dst_hbm, sidx2, didx2, sem_i)
    _issue_gather(0, off, src_hbm, xl_hbm, xr_hbm, sidx2, didx2, idx_sg2,
                  idx_dg2, xlb2, xrb2, sem_i, sem_g)

    def body(u, _):
        p3 = lax.rem(u, 3)
        p2 = lax.rem(u, 2)
        for j in range(CH // 16):
            idx_sc3[pl.ds(p3 * CH + j * 16, 16)] = (
                didx2[pl.ds(p2 * CH + j * 16, 16)])

        t3 = lax.rem(u + 1, 3)

        @pl.when(jnp.logical_and(u + 1 < nch, u >= 2))
        def _():
            _wait_scatter(t3, wbuf3, idx_sc3, esc3, denS, outS, sem_sc)

        @pl.when(u + 1 < nch)
        def _():
            _issue_gather(u + 1, off, src_hbm, xl_hbm, xr_hbm, sidx2, didx2,
                          idx_sg2, idx_dg2, xlb2, xrb2, sem_i, sem_g)

        @pl.when(u + 2 < nch)
        def _():
            _issue_idx(u + 2, ebase, src_hbm, dst_hbm, sidx2, didx2, sem_i)

        pltpu.make_async_copy(xl_hbm.at[pl.ds(0, CH)],
                              xlb2.at[pl.ds(p2 * CH, CH)], sem_g.at[p2]).wait()
        pltpu.make_async_copy(xl_hbm.at[pl.ds(0, CH)],
                              xrb2.at[pl.ds(p2 * CH, CH)], sem_g.at[p2]).wait()
        _compute_chunk(p3, p2, alin, aabs, xlb2, xrb2, wbuf3, tbf, esc3)
        pltpu.async_copy(wbuf3.at[pl.ds(p3 * CH, CH)],
                         outS.at[idx_sc3.at[pl.ds(p3 * CH, CH)]],
                         sem_sc.at[p3], add=True)
        pltpu.async_copy(esc3.at[pl.ds(p3 * CH, CH)],
                         denS.at[idx_sc3.at[pl.ds(p3 * CH, CH)]],
                         sem_sc.at[p3], add=True)
        return 0

    lax.fori_loop(0, nch, body, 0)
    _wait_scatter(lax.rem(nch - 2, 3), wbuf3, idx_sc3, esc3, denS, outS, sem_sc)
    _wait_scatter(lax.rem(nch - 1, 3), wbuf3, idx_sc3, esc3, denS, outS, sem_sc)


_SC_SCRATCH = dict(
    sidx2=pltpu.VMEM((2 * CH,), jnp.int32),
    didx2=pltpu.VMEM((2 * CH,), jnp.int32),
    idx_sg2=pltpu.VMEM((2 * CH,), jnp.int32),
    idx_dg2=pltpu.VMEM((2 * CH,), jnp.int32),
    idx_sc3=pltpu.VMEM((3 * CH,), jnp.int32),
    xlb2=pltpu.VMEM((2 * CH, C // 2), jnp.int32),
    xrb2=pltpu.VMEM((2 * CH, C // 2), jnp.int32),
    wbuf3=pltpu.VMEM((3 * CH, C), jnp.float32),
    esc3=pltpu.VMEM((3 * CH,), jnp.float32),
    tbf=pltpu.VMEM((256,), jnp.float32),
    attr=pltpu.VMEM((1, C), jnp.float32),
    dstr=pltpu.VMEM((NPT,), jnp.float32),
    sem_i=pltpu.SemaphoreType.DMA((2,)),
    sem_g=pltpu.SemaphoreType.DMA((2,)),
    sem_sc=pltpu.SemaphoreType.DMA((3,)),
    denS=pltpu.VMEM_SHARED((NP,), jnp.float32),
    outS=pltpu.VMEM_SHARED((NP, C), jnp.float32),
)


@functools.partial(
    pl.kernel,
    out_type=jax.ShapeDtypeStruct((H * NP, C), jnp.float32),
    mesh=_mesh,
    compiler_params=pltpu.CompilerParams(needs_layout_passes=False, use_tc_tiling_on_sc=False),
    scratch_types=dict(_SC_SCRATCH),
)
def _l1_edges(xl_hbm, xr_hbm, src_hbm, dst_hbm, att_hbm, out_hbm, *,
              sidx2, didx2, idx_sg2, idx_dg2, idx_sc3, xlb2, xrb2, wbuf3,
              esc3, tbf, attr, dstr, sem_i, sem_g, sem_sc, denS, outS):
    c = lax.axis_index("c")
    s = lax.axis_index("s")
    zi = jnp.zeros((16,), jnp.int32)

    def head(hl, _):
        hg = c * 4 + hl
        off = hg * NP
        pltpu.sync_copy(att_hbm.at[pl.ds(hg, 1)], attr)
        alin, aabs = _att_vecs(attr)
        _zero_flat(dstr, NPT)
        _zero_rows(wbuf3, CH, C)
        pltpu.sync_copy(dstr, denS.at[pl.ds(s * NPT, NPT)])
        for r in range(NPT // CH):
            pltpu.sync_copy(wbuf3.at[pl.ds(0, CH)],
                            outS.at[pl.ds(s * NPT + r * CH, CH)])
        plsc.subcore_barrier()

        _edge_pass(NCH1, s * EPT1, off, src_hbm, dst_hbm, xl_hbm, xr_hbm,
                   alin, aabs, sidx2, didx2, idx_sg2, idx_dg2, idx_sc3,
                   xlb2, xrb2, wbuf3, tbf, esc3, sem_i, sem_g, sem_sc,
                   denS, outS)
        plsc.subcore_barrier()

        pltpu.sync_copy(denS.at[pl.ds(s * NPT, NPT)], dstr)
        for rr in range(NPT // CH):
            pltpu.sync_copy(outS.at[pl.ds(s * NPT + rr * CH, CH)],
                            wbuf3.at[pl.ds(0, CH)])

            def norm(k, _):
                dv = plsc.load_gather(dstr, [zi + (rr * CH + k)])
                w = 1.0 / (dv + 1e-16)
                for j in range(C // 16):
                    wbuf3[k, pl.ds(j * 16, 16)] = wbuf3[k, pl.ds(j * 16, 16)] * w
                return 0

            lax.fori_loop(0, CH, norm, 0)
            pltpu.sync_copy(wbuf3.at[pl.ds(0, CH)],
                            out_hbm.at[pl.ds(off + s * NPT + rr * CH, CH)])
        plsc.subcore_barrier()
        return 0

    lax.fori_loop(0, 4, head, 0)


@functools.partial(
    pl.kernel,
    out_type=[
        jax.ShapeDtypeStruct((2 * NP, C), jnp.float32),
        jax.ShapeDtypeStruct((1, 2 * NP), jnp.float32),
    ],
    mesh=_mesh,
    compiler_params=pltpu.CompilerParams(needs_layout_passes=False, use_tc_tiling_on_sc=False),
    scratch_types=dict(_SC_SCRATCH),
)
def _l2_edges(xl_hbm, xr_hbm, src_hbm, dst_hbm, att_hbm, num_hbm, den_hbm, *,
              sidx2, didx2, idx_sg2, idx_dg2, idx_sc3, xlb2, xrb2, wbuf3,
              esc3, tbf, attr, dstr, sem_i, sem_g, sem_sc, denS, outS):
    c = lax.axis_index("c")
    s = lax.axis_index("s")
    wid = c * 16 + s
    pltpu.sync_copy(att_hbm, attr)
    alin, aabs = _att_vecs(attr)
    _zero_flat(dstr, NPT)
    _zero_rows(wbuf3, CH, C)
    pltpu.sync_copy(dstr, denS.at[pl.ds(s * NPT, NPT)])
    for r in range(NPT // CH):
        pltpu.sync_copy(wbuf3.at[pl.ds(0, CH)],
                        outS.at[pl.ds(s * NPT + r * CH, CH)])
    plsc.subcore_barrier()

    _edge_pass(NCH2, wid * EPT2, 0, src_hbm, dst_hbm, xl_hbm, xr_hbm,
               alin, aabs, sidx2, didx2, idx_sg2, idx_dg2, idx_sc3,
               xlb2, xrb2, wbuf3, tbf, esc3, sem_i, sem_g, sem_sc, denS, outS)
    plsc.subcore_barrier()
    pltpu.sync_copy(outS.at[pl.ds(s * NPT, NPT)],
                    num_hbm.at[pl.ds(c * NP + s * NPT, NPT)])
    pltpu.sync_copy(denS.at[pl.ds(s * NPT, NPT)],
                    den_hbm.at[0, pl.ds(c * NP + s * NPT, NPT)])


def kernel(x, edge_index, batch, Wl1, bl1, Wr1, br1, att1, bias1,
           Wl2, bl2, Wr2, br2, att2, bias2, Wlin, blin):
    xp = jnp.pad(x, ((0, NP - N), (0, 0)))
    loops = jnp.arange(N, dtype=jnp.int32)
    npad = EPAD - ET
    srcp = jnp.concatenate([edge_index[0], loops,
                            jnp.zeros((npad,), jnp.int32)])
    dstp = jnp.concatenate([edge_index[1], loops,
                            jnp.full((npad,), N, jnp.int32)])
    batchp = jnp.concatenate([batch, jnp.full((NP - N,), 64, jnp.int32)]
                             ).reshape(1, NP)

    wl3 = Wl1.reshape(C, H, C).transpose(1, 0, 2)
    wr3 = Wr1.reshape(C, H, C).transpose(1, 0, 2)
    bl3 = bl1.reshape(H, 1, C)
    br3 = br1.reshape(H, 1, C)
    xlf, xrf = _proj1(xp, wl3, wr3, bl3, br3)

    h1f = _l1_edges(xlf, xrf, srcp, dstp, att1)

    b13 = bias1.reshape(H, 1, C)
    wl23 = Wl2.reshape(H, C, C)
    wr23 = Wr2.reshape(H, C, C)
    xl2, xr2 = _proj2(h1f, b13, wl23, wr23, bl2.reshape(1, C), br2.reshape(1, C))

    numpf, denpf = _l2_edges(xl2, xr2, srcp, dstp, att2)

    wlinp = jnp.pad(Wlin, ((0, 0), (0, C - 40)))
    blinp = jnp.pad(blin, ((0, C - 40),)).reshape(1, C)
    b2r = bias2.reshape(1, C)
    out = _pool(numpf, denpf, b2r, batchp, wlinp, blinp)
    return out[:, :40]


# speedup vs baseline: 1.2174x; 1.2174x over previous
import functools

import jax
import jax.numpy as jnp
from jax import lax
from jax.experimental import pallas as pl
from jax.experimental.pallas import tpu as pltpu
from jax.experimental.pallas import tpu_sc as plsc

N = 10000
NP = 10240
NPT = NP // 16
E = 320000
ET = E + N
EPAD = 331776
H = 8
C = 128
CH = 64
EPT1 = EPAD // 16
NCH1 = EPT1 // CH
EPT2 = EPAD // 32
NCH2 = EPT2 // CH
NB = NP // 256
NEG = 0.2
A_LIN = (1.0 + NEG) / 2.0
A_ABS = (1.0 - NEG) / 2.0

_mesh = plsc.VectorSubcoreMesh(core_axis_name="c", subcore_axis_name="s")


def _proj1_body(x_ref, wl_ref, wr_ref, bl_ref, br_ref, xl_ref, xr_ref):
    x = x_ref[...]
    xl_ref[...] = jnp.dot(x, wl_ref[0], preferred_element_type=jnp.float32) + bl_ref[0]
    xr_ref[...] = jnp.dot(x, wr_ref[0], preferred_element_type=jnp.float32) + br_ref[0]


def _proj1(xp, wl3, wr3, bl3, br3):
    return pl.pallas_call(
        _proj1_body,
        grid=(H, NB),
        in_specs=[
            pl.BlockSpec((256, C), lambda h, n: (n, 0)),
            pl.BlockSpec((1, C, C), lambda h, n: (h, 0, 0)),
            pl.BlockSpec((1, C, C), lambda h, n: (h, 0, 0)),
            pl.BlockSpec((1, 1, C), lambda h, n: (h, 0, 0)),
            pl.BlockSpec((1, 1, C), lambda h, n: (h, 0, 0)),
        ],
        out_specs=[
            pl.BlockSpec((256, C), lambda h, n: (h * NB + n, 0)),
            pl.BlockSpec((256, C), lambda h, n: (h * NB + n, 0)),
        ],
        out_shape=[
            jax.ShapeDtypeStruct((H * NP, C), jnp.float32),
            jax.ShapeDtypeStruct((H * NP, C), jnp.float32),
        ],
    )(xp, wl3, wr3, bl3, br3)


def _proj2_body(h1_ref, d1_ref, b1_ref, wl_ref, wr_ref, bl_ref, br_ref, xl_ref, xr_ref):
    h = pl.program_id(1)
    w = 1.0 / (d1_ref[...] + 1e-16)
    t = jnp.maximum(h1_ref[...] * w + b1_ref[0], 0.0)
    cl = jnp.dot(t, wl_ref[0], preferred_element_type=jnp.float32)
    cr = jnp.dot(t, wr_ref[0], preferred_element_type=jnp.float32)

    @pl.when(h == 0)
    def _():
        xl_ref[...] = cl + bl_ref[...]
        xr_ref[...] = cr + br_ref[...]

    @pl.when(h != 0)
    def _():
        xl_ref[...] += cl
        xr_ref[...] += cr


def _proj2(h1f, d1f, b13, wl23, wr23, bl2r, br2r):
    return pl.pallas_call(
        _proj2_body,
        grid=(NB, H),
        in_specs=[
            pl.BlockSpec((256, C), lambda n, h: (h * NB + n, 0)),
            pl.BlockSpec((256, 1), lambda n, h: (h * NB + n, 0)),
            pl.BlockSpec((1, 1, C), lambda n, h: (h, 0, 0)),
            pl.BlockSpec((1, C, C), lambda n, h: (h, 0, 0)),
            pl.BlockSpec((1, C, C), lambda n, h: (h, 0, 0)),
            pl.BlockSpec((1, C), lambda n, h: (0, 0)),
            pl.BlockSpec((1, C), lambda n, h: (0, 0)),
        ],
        out_specs=[
            pl.BlockSpec((256, C), lambda n, h: (n, 0)),
            pl.BlockSpec((256, C), lambda n, h: (n, 0)),
        ],
        out_shape=[
            jax.ShapeDtypeStruct((NP, C), jnp.float32),
            jax.ShapeDtypeStruct((NP, C), jnp.float32),
        ],
    )(h1f, d1f, b13, wl23, wr23, bl2r, br2r)


def _pool_body(nump_ref, denp_ref, b2_ref, batch_ref, wlin_ref, blin_ref, out_ref):
    num = nump_ref[:NP, :] + nump_ref[NP:, :]
    den = denp_ref[:, :NP] + denp_ref[:, NP:]
    b2d = batch_ref[...]
    iota64 = lax.broadcasted_iota(jnp.int32, (64, 1), 0)
    mf = (b2d == iota64).astype(jnp.float32)
    wf = mf / (den + 1e-16)
    sums = jnp.dot(wf, num, preferred_element_type=jnp.float32)
    counts = jnp.sum(mf, axis=1, keepdims=True)
    hg = sums / jnp.maximum(counts, 1.0) + b2_ref[...]
    logits = jnp.dot(hg, wlin_ref[...], preferred_element_type=jnp.float32) + blin_ref[...]
    colid = lax.broadcasted_iota(jnp.int32, (64, C), 1)
    valid = colid < 40
    m = jnp.max(jnp.where(valid, logits, -1e30), axis=1, keepdims=True)
    ex = jnp.where(valid, jnp.exp(logits - m), 0.0)
    lse = jnp.log(jnp.sum(ex, axis=1, keepdims=True))
    out_ref[...] = logits - m - lse


def _pool(numpf, denpf, b2r, batchp, wlinp, blinp):
    return pl.pallas_call(
        _pool_body,
        out_shape=jax.ShapeDtypeStruct((64, C), jnp.float32),
    )(numpf, denpf, b2r, batchp, wlinp, blinp)


def _zero_rows(ref, rows, cols):
    def body(i, _):
        for j in range(cols // 16):
            ref[i, pl.ds(j * 16, 16)] = jnp.zeros((16,), jnp.float32)
        return 0
    lax.fori_loop(0, rows, body, 0)


def _zero_flat(ref, n):
    def body(i, _):
        ref[pl.ds(i * 16, 16)] = jnp.zeros((16,), jnp.float32)
        return 0
    lax.fori_loop(0, n // 16, body, 0)


def _issue_idx(u, ebase, src_hbm, dst_hbm, sidx2, didx2, sem_i):
    p = lax.rem(u, 2)
    base = ebase + u * CH
    pltpu.async_copy(src_hbm.at[pl.ds(base, CH)], sidx2.at[pl.ds(p * CH, CH)],
                     sem_i.at[p])
    pltpu.async_copy(dst_hbm.at[pl.ds(base, CH)], didx2.at[pl.ds(p * CH, CH)],
                     sem_i.at[p])


def _issue_gather(u, off, src_hbm, xl_hbm, xr_hbm, sidx2, didx2, idx_sg2,
                  idx_dg2, xlr3, xrr2, sem_i, sem_g):
    p2 = lax.rem(u, 2)
    p3 = lax.rem(u, 3)
    pltpu.make_async_copy(src_hbm.at[pl.ds(0, CH)],
                          sidx2.at[pl.ds(p2 * CH, CH)], sem_i.at[p2]).wait()
    pltpu.make_async_copy(src_hbm.at[pl.ds(0, CH)],
                          didx2.at[pl.ds(p2 * CH, CH)], sem_i.at[p2]).wait()
    for j in range(CH // 16):
        idx_sg2[pl.ds(p2 * CH + j * 16, 16)] = (
            sidx2[pl.ds(p2 * CH + j * 16, 16)] + off)
        idx_dg2[pl.ds(p2 * CH + j * 16, 16)] = (
            didx2[pl.ds(p2 * CH + j * 16, 16)] + off)
    pltpu.async_copy(xl_hbm.at[idx_sg2.at[pl.ds(p2 * CH, CH)]],
                     xlr3.at[pl.ds(p3 * CH, CH)], sem_g.at[p3])
    pltpu.async_copy(xr_hbm.at[idx_dg2.at[pl.ds(p2 * CH, CH)]],
                     xrr2.at[pl.ds(p2 * CH, CH)], sem_g.at[p3])


def _compute_chunk(p3, p2, attr, xlr3, xrr2, tbf, esc3):
    pb3 = p3 * CH
    pb2 = p2 * CH
    iota = lax.iota(jnp.int32, 16)
    zi = jnp.zeros((16,), jnp.int32)

    def grp(g, _):
        accs = [jnp.zeros((16,), jnp.float32) for _ in range(16)]
        for j in range(C // 16):
            aj = attr[0, pl.ds(j * 16, 16)]
            a_lin = aj * A_LIN
            a_abs = aj * A_ABS
            for k in range(16):
                e = g * 16 + k
                z = (xlr3[pb3 + e, pl.ds(j * 16, 16)]
                     + xrr2[pb2 + e, pl.ds(j * 16, 16)])
                accs[k] = accs[k] + a_lin * z + a_abs * jnp.abs(z)
        for k in range(16):
            plsc.store_scatter(tbf, [iota * 16 + k], accs[k])
        s16 = tbf[pl.ds(0, 16)]
        for i in range(1, 16):
            s16 = s16 + tbf[pl.ds(i * 16, 16)]
        esc3[pl.ds(pb3 + g * 16, 16)] = jnp.exp(s16)
        return 0

    lax.fori_loop(0, CH // 16, grp, 0)

    def wrow(e, _):
        we = plsc.load_gather(esc3, [zi + (pb3 + e)])
        for j in range(C // 16):
            xlr3[pb3 + e, pl.ds(j * 16, 16)] = (
                xlr3[pb3 + e, pl.ds(j * 16, 16)] * we)
        return 0

    lax.fori_loop(0, CH, wrow, 0)


def _wait_scatter(slot, xlr3, idx_sc3, esc3, denS, outS, sem_sc):
    pltpu.make_async_copy(
        xlr3.at[pl.ds(slot * CH, CH)],
        outS.at[idx_sc3.at[pl.ds(slot * CH, CH)]], sem_sc.at[slot]).wait()
    pltpu.make_async_copy(
        esc3.at[pl.ds(slot * CH, CH)],
        denS.at[idx_sc3.at[pl.ds(slot * CH, CH)]], sem_sc.at[slot]).wait()


def _edge_pass(nch, ebase, off, src_hbm, dst_hbm, xl_hbm, xr_hbm, attr,
               sidx2, didx2, idx_sg2, idx_dg2, idx_sc3, xlr3, xrr2, tbf, esc3,
               sem_i, sem_g, sem_sc, denS, outS):
    _issue_idx(0, ebase, src_hbm, dst_hbm, sidx2, didx2, sem_i)
    _issue_idx(1, ebase, src_hbm, dst_hbm, sidx2, didx2, sem_i)
    _issue_gather(0, off, src_hbm, xl_hbm, xr_hbm, sidx2, didx2, idx_sg2,
                  idx_dg2, xlr3, xrr2, sem_i, sem_g)

    def body(u, _):
        p3 = lax.rem(u, 3)
        p2 = lax.rem(u, 2)
        for j in range(CH // 16):
            idx_sc3[pl.ds(p3 * CH + j * 16, 16)] = (
                didx2[pl.ds(p2 * CH + j * 16, 16)])

        t3 = lax.rem(u + 1, 3)

        @pl.when(jnp.logical_and(u + 1 < nch, u >= 2))
        def _():
            _wait_scatter(t3, xlr3, idx_sc3, esc3, denS, outS, sem_sc)

        @pl.when(u + 1 < nch)
        def _():
            _issue_gather(u + 1, off, src_hbm, xl_hbm, xr_hbm, sidx2, didx2,
                          idx_sg2, idx_dg2, xlr3, xrr2, sem_i, sem_g)

        @pl.when(u + 2 < nch)
        def _():
            _issue_idx(u + 2, ebase, src_hbm, dst_hbm, sidx2, didx2, sem_i)

        pltpu.make_async_copy(xl_hbm.at[pl.ds(0, CH)],
                              xlr3.at[pl.ds(p3 * CH, CH)], sem_g.at[p3]).wait()
        pltpu.make_async_copy(xl_hbm.at[pl.ds(0, CH)],
                              xrr2.at[pl.ds(p2 * CH, CH)], sem_g.at[p3]).wait()
        _compute_chunk(p3, p2, attr, xlr3, xrr2, tbf, esc3)
        pltpu.async_copy(xlr3.at[pl.ds(p3 * CH, CH)],
                         outS.at[idx_sc3.at[pl.ds(p3 * CH, CH)]],
                         sem_sc.at[p3], add=True)
        pltpu.async_copy(esc3.at[pl.ds(p3 * CH, CH)],
                         denS.at[idx_sc3.at[pl.ds(p3 * CH, CH)]],
                         sem_sc.at[p3], add=True)
        return 0

    lax.fori_loop(0, nch, body, 0)
    _wait_scatter(lax.rem(nch - 2, 3), xlr3, idx_sc3, esc3, denS, outS, sem_sc)
    _wait_scatter(lax.rem(nch - 1, 3), xlr3, idx_sc3, esc3, denS, outS, sem_sc)


_SC_SCRATCH = dict(
    sidx2=pltpu.VMEM((2 * CH,), jnp.int32),
    didx2=pltpu.VMEM((2 * CH,), jnp.int32),
    idx_sg2=pltpu.VMEM((2 * CH,), jnp.int32),
    idx_dg2=pltpu.VMEM((2 * CH,), jnp.int32),
    idx_sc3=pltpu.VMEM((3 * CH,), jnp.int32),
    xlr3=pltpu.VMEM((3 * CH, C), jnp.float32),
    xrr2=pltpu.VMEM((2 * CH, C), jnp.float32),
    esc3=pltpu.VMEM((3 * CH,), jnp.float32),
    tbf=pltpu.VMEM((256,), jnp.float32),
    attr=pltpu.VMEM((1, C), jnp.float32),
    dstr=pltpu.VMEM((NPT,), jnp.float32),
    sem_i=pltpu.SemaphoreType.DMA((2,)),
    sem_g=pltpu.SemaphoreType.DMA((3,)),
    sem_sc=pltpu.SemaphoreType.DMA((3,)),
    denS=pltpu.VMEM_SHARED((NP,), jnp.float32),
    outS=pltpu.VMEM_SHARED((NP, C), jnp.float32),
)


@functools.partial(
    pl.kernel,
    out_type=[
        jax.ShapeDtypeStruct((H * NP, C), jnp.float32),
        jax.ShapeDtypeStruct((H * NP,), jnp.float32),
    ],
    mesh=_mesh,
    compiler_params=pltpu.CompilerParams(needs_layout_passes=False),
    scratch_types=dict(_SC_SCRATCH),
)
def _l1_edges(xl_hbm, xr_hbm, src_hbm, dst_hbm, att_hbm, out_hbm, den_hbm, *,
              sidx2, didx2, idx_sg2, idx_dg2, idx_sc3, xlr3, xrr2, esc3, tbf,
              attr, dstr, sem_i, sem_g, sem_sc, denS, outS):
    c = lax.axis_index("c")
    s = lax.axis_index("s")

    def head(hl, _):
        hg = c * 4 + hl
        off = hg * NP
        pltpu.sync_copy(att_hbm.at[pl.ds(hg, 1)], attr)
        _zero_flat(dstr, NPT)
        _zero_rows(xrr2, CH, C)
        pltpu.sync_copy(dstr, denS.at[pl.ds(s * NPT, NPT)])
        for r in range(NPT // CH):
            pltpu.sync_copy(xrr2.at[pl.ds(0, CH)],
                            outS.at[pl.ds(s * NPT + r * CH, CH)])
        plsc.subcore_barrier()

        _edge_pass(NCH1, s * EPT1, off, src_hbm, dst_hbm, xl_hbm, xr_hbm,
                   attr, sidx2, didx2, idx_sg2, idx_dg2, idx_sc3, xlr3, xrr2,
                   tbf, esc3, sem_i, sem_g, sem_sc, denS, outS)
        plsc.subcore_barrier()

        pltpu.sync_copy(outS.at[pl.ds(s * NPT, NPT)],
                        out_hbm.at[pl.ds(off + s * NPT, NPT)])
        pltpu.sync_copy(denS.at[pl.ds(s * NPT, NPT)],
                        den_hbm.at[pl.ds(off + s * NPT, NPT)])
        plsc.subcore_barrier()
        return 0

    lax.fori_loop(0, 4, head, 0)


@functools.partial(
    pl.kernel,
    out_type=[
        jax.ShapeDtypeStruct((2 * NP, C), jnp.float32),
        jax.ShapeDtypeStruct((1, 2 * NP), jnp.float32),
    ],
    mesh=_mesh,
    compiler_params=pltpu.CompilerParams(needs_layout_passes=False),
    scratch_types=dict(_SC_SCRATCH),
)
def _l2_edges(xl_hbm, xr_hbm, src_hbm, dst_hbm, att_hbm, num_hbm, den_hbm, *,
              sidx2, didx2, idx_sg2, idx_dg2, idx_sc3, xlr3, xrr2, esc3, tbf,
              attr, dstr, sem_i, sem_g, sem_sc, denS, outS):
    c = lax.axis_index("c")
    s = lax.axis_index("s")
    wid = c * 16 + s
    pltpu.sync_copy(att_hbm, attr)
    _zero_flat(dstr, NPT)
    _zero_rows(xrr2, CH, C)
    pltpu.sync_copy(dstr, denS.at[pl.ds(s * NPT, NPT)])
    for r in range(NPT // CH):
        pltpu.sync_copy(xrr2.at[pl.ds(0, CH)],
                        outS.at[pl.ds(s * NPT + r * CH, CH)])
    plsc.subcore_barrier()

    _edge_pass(NCH2, wid * EPT2, 0, src_hbm, dst_hbm, xl_hbm, xr_hbm,
               attr, sidx2, didx2, idx_sg2, idx_dg2, idx_sc3, xlr3, xrr2,
               tbf, esc3, sem_i, sem_g, sem_sc, denS, outS)
    plsc.subcore_barrier()
    pltpu.sync_copy(outS.at[pl.ds(s * NPT, NPT)],
                    num_hbm.at[pl.ds(c * NP + s * NPT, NPT)])
    pltpu.sync_copy(denS.at[pl.ds(s * NPT, NPT)],
                    den_hbm.at[0, pl.ds(c * NP + s * NPT, NPT)])


def kernel(x, edge_index, batch, Wl1, bl1, Wr1, br1, att1, bias1,
           Wl2, bl2, Wr2, br2, att2, bias2, Wlin, blin):
    xp = jnp.pad(x, ((0, NP - N), (0, 0)))
    loops = jnp.arange(N, dtype=jnp.int32)
    npad = EPAD - ET
    srcp = jnp.concatenate([edge_index[0], loops,
                            jnp.zeros((npad,), jnp.int32)])
    dstp = jnp.concatenate([edge_index[1], loops,
                            jnp.full((npad,), N, jnp.int32)])
    batchp = jnp.concatenate([batch, jnp.full((NP - N,), 64, jnp.int32)]
                             ).reshape(1, NP)

    wl3 = Wl1.reshape(C, H, C).transpose(1, 0, 2)
    wr3 = Wr1.reshape(C, H, C).transpose(1, 0, 2)
    bl3 = bl1.reshape(H, 1, C)
    br3 = br1.reshape(H, 1, C)
    xlf, xrf = _proj1(xp, wl3, wr3, bl3, br3)

    h1f, den1 = _l1_edges(xlf, xrf, srcp, dstp, att1)
    d1f = den1.reshape(H * NP, 1)

    b13 = bias1.reshape(H, 1, C)
    wl23 = Wl2.reshape(H, C, C)
    wr23 = Wr2.reshape(H, C, C)
    xl2, xr2 = _proj2(h1f, d1f, b13, wl23, wr23, bl2.reshape(1, C), br2.reshape(1, C))

    numpf, denpf = _l2_edges(xl2, xr2, srcp, dstp, att2)

    wlinp = jnp.pad(Wlin, ((0, 0), (0, C - 40)))
    blinp = jnp.pad(blin, ((0, C - 40),)).reshape(1, C)
    out = _pool(numpf, denpf, bias2.reshape(1, C), batchp, wlinp, blinp)
    return out[:, :40]


# speedup vs baseline: 1.2474x; 1.0247x over previous
import functools

import jax
import jax.numpy as jnp
from jax import lax
from jax.experimental import pallas as pl
from jax.experimental.pallas import tpu as pltpu
from jax.experimental.pallas import tpu_sc as plsc

N = 10000
NP = 10240
NPT = NP // 16
E = 320000
ET = E + N
EPAD = 331776
H = 8
C = 128
CH = 64
EPT1 = EPAD // 16
NCH1 = EPT1 // CH
EPT2 = EPAD // 32
NCH2 = EPT2 // CH
NB = NP // 256
NEG = 0.2
A_LIN = (1.0 + NEG) / 2.0
A_ABS = (1.0 - NEG) / 2.0

_mesh = plsc.VectorSubcoreMesh(core_axis_name="c", subcore_axis_name="s")


def _proj1_body(x_ref, wl_ref, wr_ref, bl_ref, br_ref, xl_ref, xr_ref):
    x = x_ref[...]
    xl_ref[...] = jnp.dot(x, wl_ref[0], preferred_element_type=jnp.float32) + bl_ref[0]
    xr_ref[...] = jnp.dot(x, wr_ref[0], preferred_element_type=jnp.float32) + br_ref[0]


def _proj1(xp, wl3, wr3, bl3, br3):
    return pl.pallas_call(
        _proj1_body,
        grid=(H, NB),
        in_specs=[
            pl.BlockSpec((256, C), lambda h, n: (n, 0)),
            pl.BlockSpec((1, C, C), lambda h, n: (h, 0, 0)),
            pl.BlockSpec((1, C, C), lambda h, n: (h, 0, 0)),
            pl.BlockSpec((1, 1, C), lambda h, n: (h, 0, 0)),
            pl.BlockSpec((1, 1, C), lambda h, n: (h, 0, 0)),
        ],
        out_specs=[
            pl.BlockSpec((256, C), lambda h, n: (h * NB + n, 0)),
            pl.BlockSpec((256, C), lambda h, n: (h * NB + n, 0)),
        ],
        out_shape=[
            jax.ShapeDtypeStruct((H * NP, C), jnp.float32),
            jax.ShapeDtypeStruct((H * NP, C), jnp.float32),
        ],
    )(xp, wl3, wr3, bl3, br3)


def _proj2_body(h1_ref, b1_ref, wl_ref, wr_ref, bl_ref, br_ref, xl_ref, xr_ref):
    h = pl.program_id(1)
    t = jnp.maximum(h1_ref[...] + b1_ref[0], 0.0)
    cl = jnp.dot(t, wl_ref[0], preferred_element_type=jnp.float32)
    cr = jnp.dot(t, wr_ref[0], preferred_element_type=jnp.float32)

    @pl.when(h == 0)
    def _():
        xl_ref[...] = cl + bl_ref[...]
        xr_ref[...] = cr + br_ref[...]

    @pl.when(h != 0)
    def _():
        xl_ref[...] += cl
        xr_ref[...] += cr


def _proj2(h1f, b13, wl23, wr23, bl2r, br2r):
    return pl.pallas_call(
        _proj2_body,
        grid=(NB, H),
        in_specs=[
            pl.BlockSpec((256, C), lambda n, h: (h * NB + n, 0)),
            pl.BlockSpec((1, 1, C), lambda n, h: (h, 0, 0)),
            pl.BlockSpec((1, C, C), lambda n, h: (h, 0, 0)),
            pl.BlockSpec((1, C, C), lambda n, h: (h, 0, 0)),
            pl.BlockSpec((1, C), lambda n, h: (0, 0)),
            pl.BlockSpec((1, C), lambda n, h: (0, 0)),
        ],
        out_specs=[
            pl.BlockSpec((256, C), lambda n, h: (n, 0)),
            pl.BlockSpec((256, C), lambda n, h: (n, 0)),
        ],
        out_shape=[
            jax.ShapeDtypeStruct((NP, C), jnp.float32),
            jax.ShapeDtypeStruct((NP, C), jnp.float32),
        ],
    )(h1f, b13, wl23, wr23, bl2r, br2r)


def _pool_body(nump_ref, denp_ref, b2_ref, batch_ref, wlin_ref, blin_ref, out_ref):
    num = nump_ref[:NP, :] + nump_ref[NP:, :]
    den = denp_ref[:, :NP] + denp_ref[:, NP:]
    b2d = batch_ref[...]
    iota64 = lax.broadcasted_iota(jnp.int32, (64, 1), 0)
    mf = (b2d == iota64).astype(jnp.float32)
    wf = mf / (den + 1e-16)
    sums = jnp.dot(wf, num, preferred_element_type=jnp.float32)
    counts = jnp.sum(mf, axis=1, keepdims=True)
    hg = sums / jnp.maximum(counts, 1.0) + b2_ref[...]
    logits = jnp.dot(hg, wlin_ref[...], preferred_element_type=jnp.float32) + blin_ref[...]
    colid = lax.broadcasted_iota(jnp.int32, (64, C), 1)
    valid = colid < 40
    m = jnp.max(jnp.where(valid, logits, -1e30), axis=1, keepdims=True)
    ex = jnp.where(valid, jnp.exp(logits - m), 0.0)
    lse = jnp.log(jnp.sum(ex, axis=1, keepdims=True))
    out_ref[...] = logits - m - lse


def _pool(numpf, denpf, b2r, batchp, wlinp, blinp):
    return pl.pallas_call(
        _pool_body,
        out_shape=jax.ShapeDtypeStruct((64, C), jnp.float32),
    )(numpf, denpf, b2r, batchp, wlinp, blinp)


def _zero_rows(ref, rows, cols):
    def body(i, _):
        for j in range(cols // 16):
            ref[i, pl.ds(j * 16, 16)] = jnp.zeros((16,), jnp.float32)
        return 0
    lax.fori_loop(0, rows, body, 0)


def _zero_flat(ref, n):
    def body(i, _):
        ref[pl.ds(i * 16, 16)] = jnp.zeros((16,), jnp.float32)
        return 0
    lax.fori_loop(0, n // 16, body, 0)


def _issue_idx(u, ebase, src_hbm, dst_hbm, sidx2, didx2, sem_i):
    p = lax.rem(u, 2)
    base = ebase + u * CH
    pltpu.async_copy(src_hbm.at[pl.ds(base, CH)], sidx2.at[pl.ds(p * CH, CH)],
                     sem_i.at[p])
    pltpu.async_copy(dst_hbm.at[pl.ds(base, CH)], didx2.at[pl.ds(p * CH, CH)],
                     sem_i.at[p])


def _issue_gather(u, off, src_hbm, xl_hbm, xr_hbm, sidx2, didx2, idx_sg2,
                  idx_dg2, xlr3, xrr2, sem_i, sem_g):
    p2 = lax.rem(u, 2)
    p3 = lax.rem(u, 3)
    pltpu.make_async_copy(src_hbm.at[pl.ds(0, CH)],
                          sidx2.at[pl.ds(p2 * CH, CH)], sem_i.at[p2]).wait()
    pltpu.make_async_copy(src_hbm.at[pl.ds(0, CH)],
                          didx2.at[pl.ds(p2 * CH, CH)], sem_i.at[p2]).wait()
    for j in range(CH // 16):
        idx_sg2[pl.ds(p2 * CH + j * 16, 16)] = (
            sidx2[pl.ds(p2 * CH + j * 16, 16)] + off)
        idx_dg2[pl.ds(p2 * CH + j * 16, 16)] = (
            didx2[pl.ds(p2 * CH + j * 16, 16)] + off)
    pltpu.async_copy(xl_hbm.at[idx_sg2.at[pl.ds(p2 * CH, CH)]],
                     xlr3.at[pl.ds(p3 * CH, CH)], sem_g.at[p3])
    pltpu.async_copy(xr_hbm.at[idx_dg2.at[pl.ds(p2 * CH, CH)]],
                     xrr2.at[pl.ds(p2 * CH, CH)], sem_g.at[p3])


def _compute_chunk(p3, p2, attr, xlr3, xrr2, tbf, esc3):
    pb3 = p3 * CH
    pb2 = p2 * CH
    iota = lax.iota(jnp.int32, 16)
    zi = jnp.zeros((16,), jnp.int32)

    def grp(g, _):
        accs = [jnp.zeros((16,), jnp.float32) for _ in range(16)]
        for j in range(C // 16):
            aj = attr[0, pl.ds(j * 16, 16)]
            a_lin = aj * A_LIN
            a_abs = aj * A_ABS
            for k in range(16):
                e = g * 16 + k
                z = (xlr3[pb3 + e, pl.ds(j * 16, 16)]
                     + xrr2[pb2 + e, pl.ds(j * 16, 16)])
                accs[k] = accs[k] + a_lin * z + a_abs * jnp.abs(z)
        for k in range(16):
            plsc.store_scatter(tbf, [iota * 16 + k], accs[k])
        s16 = tbf[pl.ds(0, 16)]
        for i in range(1, 16):
            s16 = s16 + tbf[pl.ds(i * 16, 16)]
        esc3[pl.ds(pb3 + g * 16, 16)] = jnp.exp(s16)
        return 0

    lax.fori_loop(0, CH // 16, grp, 0)

    def wrow(e, _):
        we = plsc.load_gather(esc3, [zi + (pb3 + e)])
        for j in range(C // 16):
            xlr3[pb3 + e, pl.ds(j * 16, 16)] = (
                xlr3[pb3 + e, pl.ds(j * 16, 16)] * we)
        return 0

    lax.fori_loop(0, CH, wrow, 0)


def _wait_scatter(slot, xlr3, idx_sc3, esc3, denS, outS, sem_sc):
    pltpu.make_async_copy(
        xlr3.at[pl.ds(slot * CH, CH)],
        outS.at[idx_sc3.at[pl.ds(slot * CH, CH)]], sem_sc.at[slot]).wait()
    pltpu.make_async_copy(
        esc3.at[pl.ds(slot * CH, CH)],
        denS.at[idx_sc3.at[pl.ds(slot * CH, CH)]], sem_sc.at[slot]).wait()


def _edge_pass(nch, ebase, off, src_hbm, dst_hbm, xl_hbm, xr_hbm, attr,
               sidx2, didx2, idx_sg2, idx_dg2, idx_sc3, xlr3, xrr2, tbf, esc3,
               sem_i, sem_g, sem_sc, denS, outS):
    _issue_idx(0, ebase, src_hbm, dst_hbm, sidx2, didx2, sem_i)
    _issue_idx(1, ebase, src_hbm, dst_hbm, sidx2, didx2, sem_i)
    _issue_gather(0, off, src_hbm, xl_hbm, xr_hbm, sidx2, didx2, idx_sg2,
                  idx_dg2, xlr3, xrr2, sem_i, sem_g)

    def body(u, _):
        p3 = lax.rem(u, 3)
        p2 = lax.rem(u, 2)
        for j in range(CH // 16):
            idx_sc3[pl.ds(p3 * CH + j * 16, 16)] = (
                didx2[pl.ds(p2 * CH + j * 16, 16)])

        t3 = lax.rem(u + 1, 3)

        @pl.when(jnp.logical_and(u + 1 < nch, u >= 2))
        def _():
            _wait_scatter(t3, xlr3, idx_sc3, esc3, denS, outS, sem_sc)

        @pl.when(u + 1 < nch)
        def _():
            _issue_gather(u + 1, off, src_hbm, xl_hbm, xr_hbm, sidx2, didx2,
                          idx_sg2, idx_dg2, xlr3, xrr2, sem_i, sem_g)

        @pl.when(u + 2 < nch)
        def _():
            _issue_idx(u + 2, ebase, src_hbm, dst_hbm, sidx2, didx2, sem_i)

        pltpu.make_async_copy(xl_hbm.at[pl.ds(0, CH)],
                              xlr3.at[pl.ds(p3 * CH, CH)], sem_g.at[p3]).wait()
        pltpu.make_async_copy(xl_hbm.at[pl.ds(0, CH)],
                              xrr2.at[pl.ds(p2 * CH, CH)], sem_g.at[p3]).wait()
        _compute_chunk(p3, p2, attr, xlr3, xrr2, tbf, esc3)
        pltpu.async_copy(xlr3.at[pl.ds(p3 * CH, CH)],
                         outS.at[idx_sc3.at[pl.ds(p3 * CH, CH)]],
                         sem_sc.at[p3], add=True)
        pltpu.async_copy(esc3.at[pl.ds(p3 * CH, CH)],
                         denS.at[idx_sc3.at[pl.ds(p3 * CH, CH)]],
                         sem_sc.at[p3], add=True)
        return 0

    lax.fori_loop(0, nch, body, 0)
    _wait_scatter(lax.rem(nch - 2, 3), xlr3, idx_sc3, esc3, denS, outS, sem_sc)
    _wait_scatter(lax.rem(nch - 1, 3), xlr3, idx_sc3, esc3, denS, outS, sem_sc)


_SC_SCRATCH = dict(
    sidx2=pltpu.VMEM((2 * CH,), jnp.int32),
    didx2=pltpu.VMEM((2 * CH,), jnp.int32),
    idx_sg2=pltpu.VMEM((2 * CH,), jnp.int32),
    idx_dg2=pltpu.VMEM((2 * CH,), jnp.int32),
    idx_sc3=pltpu.VMEM((3 * CH,), jnp.int32),
    xlr3=pltpu.VMEM((3 * CH, C), jnp.float32),
    xrr2=pltpu.VMEM((2 * CH, C), jnp.float32),
    esc3=pltpu.VMEM((3 * CH,), jnp.float32),
    tbf=pltpu.VMEM((256,), jnp.float32),
    attr=pltpu.VMEM((1, C), jnp.float32),
    dstr=pltpu.VMEM((NPT,), jnp.float32),
    sem_i=pltpu.SemaphoreType.DMA((2,)),
    sem_g=pltpu.SemaphoreType.DMA((3,)),
    sem_sc=pltpu.SemaphoreType.DMA((3,)),
    denS=pltpu.VMEM_SHARED((NP,), jnp.float32),
    outS=pltpu.VMEM_SHARED((NP, C), jnp.float32),
)


@functools.partial(
    pl.kernel,
    out_type=jax.ShapeDtypeStruct((H * NP, C), jnp.float32),
    mesh=_mesh,
    compiler_params=pltpu.CompilerParams(needs_layout_passes=False),
    scratch_types=dict(_SC_SCRATCH),
)
def _l1_edges(xl_hbm, xr_hbm, src_hbm, dst_hbm, att_hbm, out_hbm, *,
              sidx2, didx2, idx_sg2, idx_dg2, idx_sc3, xlr3, xrr2, esc3, tbf,
              attr, dstr, sem_i, sem_g, sem_sc, denS, outS):
    c = lax.axis_index("c")
    s = lax.axis_index("s")
    zi = jnp.zeros((16,), jnp.int32)

    def head(hl, _):
        hg = c * 4 + hl
        off = hg * NP
        pltpu.sync_copy(att_hbm.at[pl.ds(hg, 1)], attr)
        _zero_flat(dstr, NPT)
        _zero_rows(xrr2, CH, C)
        pltpu.sync_copy(dstr, denS.at[pl.ds(s * NPT, NPT)])
        for r in range(NPT // CH):
            pltpu.sync_copy(xrr2.at[pl.ds(0, CH)],
                            outS.at[pl.ds(s * NPT + r * CH, CH)])
        plsc.subcore_barrier()

        _edge_pass(NCH1, s * EPT1, off, src_hbm, dst_hbm, xl_hbm, xr_hbm,
                   attr, sidx2, didx2, idx_sg2, idx_dg2, idx_sc3, xlr3, xrr2,
                   tbf, esc3, sem_i, sem_g, sem_sc, denS, outS)
        plsc.subcore_barrier()

        pltpu.sync_copy(denS.at[pl.ds(s * NPT, NPT)], dstr)
        for rr in range(NPT // CH):
            pltpu.sync_copy(outS.at[pl.ds(s * NPT + rr * CH, CH)],
                            xlr3.at[pl.ds(0, CH)])

            def norm(k, _):
                dv = plsc.load_gather(dstr, [zi + (rr * CH + k)])
                w = 1.0 / (dv + 1e-16)
                for j in range(C // 16):
                    xlr3[k, pl.ds(j * 16, 16)] = xlr3[k, pl.ds(j * 16, 16)] * w
                return 0

            lax.fori_loop(0, CH, norm, 0)
            pltpu.sync_copy(xlr3.at[pl.ds(0, CH)],
                            out_hbm.at[pl.ds(off + s * NPT + rr * CH, CH)])
        plsc.subcore_barrier()
        return 0

    lax.fori_loop(0, 4, head, 0)


@functools.partial(
    pl.kernel,
    out_type=[
        jax.ShapeDtypeStruct((2 * NP, C), jnp.float32),
        jax.ShapeDtypeStruct((1, 2 * NP), jnp.float32),
    ],
    mesh=_mesh,
    compiler_params=pltpu.CompilerParams(needs_layout_passes=False),
    scratch_types=dict(_SC_SCRATCH),
)
def _l2_edges(xl_hbm, xr_hbm, src_hbm, dst_hbm, att_hbm, num_hbm, den_hbm, *,
              sidx2, didx2, idx_sg2, idx_dg2, idx_sc3, xlr3, xrr2, esc3, tbf,
              attr, dstr, sem_i, sem_g, sem_sc, denS, outS):
    c = lax.axis_index("c")
    s = lax.axis_index("s")
    wid = c * 16 + s
    pltpu.sync_copy(att_hbm, attr)
    _zero_flat(dstr, NPT)
    _zero_rows(xrr2, CH, C)
    pltpu.sync_copy(dstr, denS.at[pl.ds(s * NPT, NPT)])
    for r in range(NPT // CH):
        pltpu.sync_copy(xrr2.at[pl.ds(0, CH)],
                        outS.at[pl.ds(s * NPT + r * CH, CH)])
    plsc.subcore_barrier()

    _edge_pass(NCH2, wid * EPT2, 0, src_hbm, dst_hbm, xl_hbm, xr_hbm,
               attr, sidx2, didx2, idx_sg2, idx_dg2, idx_sc3, xlr3, xrr2,
               tbf, esc3, sem_i, sem_g, sem_sc, denS, outS)
    plsc.subcore_barrier()
    pltpu.sync_copy(outS.at[pl.ds(s * NPT, NPT)],
                    num_hbm.at[pl.ds(c * NP + s * NPT, NPT)])
    pltpu.sync_copy(denS.at[pl.ds(s * NPT, NPT)],
                    den_hbm.at[0, pl.ds(c * NP + s * NPT, NPT)])


def kernel(x, edge_index, batch, Wl1, bl1, Wr1, br1, att1, bias1,
           Wl2, bl2, Wr2, br2, att2, bias2, Wlin, blin):
    xp = jnp.pad(x, ((0, NP - N), (0, 0)))
    loops = jnp.arange(N, dtype=jnp.int32)
    npad = EPAD - ET
    srcp = jnp.concatenate([edge_index[0], loops,
                            jnp.zeros((npad,), jnp.int32)])
    dstp = jnp.concatenate([edge_index[1], loops,
                            jnp.full((npad,), N, jnp.int32)])
    batchp = jnp.concatenate([batch, jnp.full((NP - N,), 64, jnp.int32)]
                             ).reshape(1, NP)

    wl3 = Wl1.reshape(C, H, C).transpose(1, 0, 2)
    wr3 = Wr1.reshape(C, H, C).transpose(1, 0, 2)
    bl3 = bl1.reshape(H, 1, C)
    br3 = br1.reshape(H, 1, C)
    xlf, xrf = _proj1(xp, wl3, wr3, bl3, br3)

    h1f = _l1_edges(xlf, xrf, srcp, dstp, att1)

    b13 = bias1.reshape(H, 1, C)
    wl23 = Wl2.reshape(H, C, C)
    wr23 = Wr2.reshape(H, C, C)
    xl2, xr2 = _proj2(h1f, b13, wl23, wr23, bl2.reshape(1, C), br2.reshape(1, C))

    numpf, denpf = _l2_edges(xl2, xr2, srcp, dstp, att2)

    wlinp = jnp.pad(Wlin, ((0, 0), (0, C - 40)))
    blinp = jnp.pad(blin, ((0, C - 40),)).reshape(1, C)
    out = _pool(numpf, denpf, bias2.reshape(1, C), batchp, wlinp, blinp)
    return out[:, :40]
